# baseline (speedup 1.0000x reference)
import functools

import numpy as np
import jax
import jax.numpy as jnp
from jax import lax
from jax.experimental import pallas as pl
from jax.experimental.pallas import tpu as pltpu
from jax.experimental.pallas import tpu_sc as plsc

NC = 2
NS = 16
L = 16
NW = NC * NS

HI_MASK = -65536


def _make_sc_kernel(B, V, scales):
  ROWS = B * V
  assert ROWS % NW == 0
  rpw = ROWS // NW
  assert rpw % L == 0
  n_chunks = rpw // L
  C_TOT = sum(c for (_, _, c, _) in scales)

  mesh = plsc.VectorSubcoreMesh(
      core_axis_name="c", subcore_axis_name="s",
      num_cores=NC, num_subcores=NS)

  scratch = [
      pltpu.VMEM((rpw,), jnp.float32),
      pltpu.VMEM((rpw,), jnp.float32),
      [pltpu.VMEM((4 * L,), jnp.int32) for _ in scales],
      [pltpu.VMEM((4 * L, c // 2), jnp.int32) for (_, _, c, _) in scales],
      [pltpu.VMEM((L, C_TOT), jnp.float32) for _ in range(2)],
      [pltpu.SemaphoreType.DMA for _ in scales],
      [pltpu.SemaphoreType.DMA for _ in range(2)],
  ]

  @functools.partial(
      pl.kernel,
      mesh=mesh,
      out_type=jax.ShapeDtypeStruct((ROWS, C_TOT), jnp.float32),
      scratch_types=scratch,
      compiler_params=pltpu.CompilerParams(needs_layout_passes=False),
  )
  def k(cx_hbm, cy_hbm, t0_hbm, t1_hbm, t2_hbm, out_hbm,
        cx_v, cy_v, idx_vs, gbufs, obufs, gsems, osems):
    tables = (t0_hbm, t1_hbm, t2_hbm)
    wid = lax.axis_index("s") * NC + lax.axis_index("c")
    base = wid * rpw
    batch = base // V

    pltpu.sync_copy(cx_hbm.at[pl.ds(base, rpw)], cx_v)
    pltpu.sync_copy(cy_hbm.at[pl.ds(base, rpw)], cy_v)

    def corner_geom(ch, s):
      H, W, C, inv = scales[s]
      x = cx_v[pl.ds(ch * L, L)] * inv
      y = cy_v[pl.ds(ch * L, L)] * inv
      x1i = x.astype(jnp.int32)
      y1i = y.astype(jnp.int32)
      x1f = x1i.astype(jnp.float32)
      y1f = y1i.astype(jnp.float32)
      one = jnp.full((L,), 1, jnp.int32)
      zero = jnp.full((L,), 0, jnp.int32)
      x2i = x1i + jnp.where(x > x1f, one, zero)
      y2i = y1i + jnp.where(y > y1f, one, zero)
      return x, y, x1i, y1i, x1f, y1f, x2i, y2i

    def fire(ch, s):
      H, W, C, inv = scales[s]
      _, _, x1i, y1i, _, _, x2i, y2i = corner_geom(ch, s)
      idx_v = idx_vs[s]
      r1 = y1i * W + batch * (H * W)
      r2 = y2i * W + batch * (H * W)
      idx_v[pl.ds(0 * L, L)] = r1 + x1i
      idx_v[pl.ds(1 * L, L)] = r2 + x1i
      idx_v[pl.ds(2 * L, L)] = r1 + x2i
      idx_v[pl.ds(3 * L, L)] = r2 + x2i
      return pltpu.async_copy(tables[s].at[idx_v], gbufs[s], gsems[s])

    dn = lax.GatherDimensionNumbers(
        offset_dims=(), collapsed_slice_dims=(0,), start_index_map=(0,))

    def splat(vec, sp):
      return lax.gather(vec, sp[:, None], dn, (1,),
                        mode=lax.GatherScatterMode.PROMISE_IN_BOUNDS)

    def compute(ch, s, coff, obuf):
      H, W, C, inv = scales[s]
      x, y, _, _, x1f, y1f, x2i, y2i = corner_geom(ch, s)
      wx2 = x - x1f
      wx1 = x2i.astype(jnp.float32) - x
      wy2 = y - y1f
      wy1 = y2i.astype(jnp.float32) - y
      w11 = wx1 * wy1
      w12 = wx1 * wy2
      w21 = wx2 * wy1
      w22 = wx2 * wy2
      gbuf = gbufs[s]

      @plsc.parallel_loop(0, L)
      def row_body(r):
        sp = jnp.full((L,), 0, jnp.int32) + r
        w0 = splat(w11, sp)
        w1 = splat(w12, sp)
        w2 = splat(w21, sp)
        w3 = splat(w22, sp)

        @plsc.parallel_loop(0, C // (2 * L), unroll=4)
        def ch_body(j):
          v0 = gbuf[0 * L + r, pl.ds(j * L, L)]
          v1 = gbuf[1 * L + r, pl.ds(j * L, L)]
          v2 = gbuf[2 * L + r, pl.ds(j * L, L)]
          v3 = gbuf[3 * L + r, pl.ds(j * L, L)]
          lo = w0 * plsc.bitcast(lax.shift_left(v0, 16), jnp.float32)
          lo += w1 * plsc.bitcast(lax.shift_left(v1, 16), jnp.float32)
          lo += w2 * plsc.bitcast(lax.shift_left(v2, 16), jnp.float32)
          lo += w3 * plsc.bitcast(lax.shift_left(v3, 16), jnp.float32)
          hi = w0 * plsc.bitcast(v0 & HI_MASK, jnp.float32)
          hi += w1 * plsc.bitcast(v1 & HI_MASK, jnp.float32)
          hi += w2 * plsc.bitcast(v2 & HI_MASK, jnp.float32)
          hi += w3 * plsc.bitcast(v3 & HI_MASK, jnp.float32)
          obuf[r, pl.ds(coff + 2 * j * L, L)] = lo
          obuf[r, pl.ds(coff + 2 * j * L + L, L)] = hi

    handles = [fire(0, s) for s in range(len(scales))]
    owrites = [None, None]
    for ch in range(n_chunks):
      slot = ch % 2
      if owrites[slot] is not None:
        owrites[slot].wait()
      obuf = obufs[slot]
      coff = 0
      for s in range(len(scales)):
        handles[s].wait()
        compute(ch, s, coff, obuf)
        if ch + 1 < n_chunks:
          handles[s] = fire(ch + 1, s)
        coff += scales[s][2]
      owrites[slot] = pltpu.async_copy(
          obuf, out_hbm.at[pl.ds(base + ch * L, L)], osems[slot])
    for ow in owrites:
      if ow is not None:
        ow.wait()

  return k


def _interleave_perm(C):
  perm = np.empty((C,), np.int64)
  for g in range(C // 32):
    base = 32 * g
    perm[base + 0:base + 32:2] = np.arange(base, base + 16)
    perm[base + 1:base + 32:2] = np.arange(base + 16, base + 32)
  return perm


def kernel(c, conv_3_3, conv_4_3, conv_5_3):
  B, V, _ = c.shape
  maps = (conv_3_3, conv_4_3, conv_5_3)
  scales = []
  inv = 1.0 / 8.0
  for fm in maps:
    _, C, H, W = fm.shape
    scales.append((H, W, C, inv))
    inv *= 0.5

  cx = c[:, :, 0].reshape(-1)
  cy = c[:, :, 1].reshape(-1)
  tables = []
  for fm in maps:
    C = fm.shape[1]
    t = fm.transpose(0, 2, 3, 1).reshape(-1, C)
    t = t.astype(jnp.bfloat16)[:, _interleave_perm(C)]
    tables.append(
        lax.bitcast_convert_type(t.reshape(-1, C // 2, 2), jnp.int32))

  k = _make_sc_kernel(B, V, tuple(scales))
  out = k(cx, cy, *tables)
  return out.reshape(B, V, out.shape[-1])


# speedup vs baseline: 1.4989x; 1.4989x over previous
import functools

import numpy as np
import jax
import jax.numpy as jnp
from jax import lax
from jax.experimental import pallas as pl
from jax.experimental.pallas import tpu as pltpu
from jax.experimental.pallas import tpu_sc as plsc

NC = 2
NS = 16
L = 16
NW = NC * NS

HI_MASK = -65536


def _make_sc_kernel(B, V, scales):
  ROWS = B * V
  assert ROWS % NW == 0
  rpw = ROWS // NW
  assert rpw % L == 0
  n_chunks = rpw // L
  C_TOT = sum(c for (_, _, c, _) in scales)

  mesh = plsc.VectorSubcoreMesh(
      core_axis_name="c", subcore_axis_name="s",
      num_cores=NC, num_subcores=NS)

  scratch = [
      pltpu.VMEM((rpw,), jnp.float32),
      pltpu.VMEM((rpw,), jnp.float32),
      [pltpu.VMEM((4 * L,), jnp.int32) for _ in scales],
      [pltpu.VMEM((4 * L, c // 2), jnp.int32) for (_, _, c, _) in scales],
      [pltpu.VMEM((L, C_TOT), jnp.float32) for _ in range(2)],
      [pltpu.SemaphoreType.DMA for _ in scales],
      [pltpu.SemaphoreType.DMA for _ in range(2)],
  ]

  @functools.partial(
      pl.kernel,
      mesh=mesh,
      out_type=jax.ShapeDtypeStruct((ROWS, C_TOT), jnp.float32),
      scratch_types=scratch,
      compiler_params=pltpu.CompilerParams(needs_layout_passes=False),
  )
  def k(cx_hbm, cy_hbm, t0_hbm, t1_hbm, t2_hbm, out_hbm,
        cx_v, cy_v, idx_vs, gbufs, obufs, gsems, osems):
    tables = (t0_hbm, t1_hbm, t2_hbm)
    wid = lax.axis_index("s") * NC + lax.axis_index("c")
    base = wid * rpw
    batch = base // V

    pltpu.sync_copy(cx_hbm.at[pl.ds(base, rpw)], cx_v)
    pltpu.sync_copy(cy_hbm.at[pl.ds(base, rpw)], cy_v)

    def corner_geom(ch, s):
      H, W, C, inv = scales[s]
      x = cx_v[pl.ds(ch * L, L)] * inv
      y = cy_v[pl.ds(ch * L, L)] * inv
      x1i = x.astype(jnp.int32)
      y1i = y.astype(jnp.int32)
      x1f = x1i.astype(jnp.float32)
      y1f = y1i.astype(jnp.float32)
      one = jnp.full((L,), 1, jnp.int32)
      zero = jnp.full((L,), 0, jnp.int32)
      x2i = x1i + jnp.where(x > x1f, one, zero)
      y2i = y1i + jnp.where(y > y1f, one, zero)
      return x, y, x1i, y1i, x1f, y1f, x2i, y2i

    def fire(ch, s):
      H, W, C, inv = scales[s]
      _, _, x1i, y1i, _, _, x2i, y2i = corner_geom(ch, s)
      idx_v = idx_vs[s]
      r1 = y1i * W + batch * (H * W)
      r2 = y2i * W + batch * (H * W)
      idx_v[pl.ds(0 * L, L)] = r1 + x1i
      idx_v[pl.ds(1 * L, L)] = r2 + x1i
      idx_v[pl.ds(2 * L, L)] = r1 + x2i
      idx_v[pl.ds(3 * L, L)] = r2 + x2i
      return pltpu.async_copy(tables[s].at[idx_v], gbufs[s], gsems[s])

    dn = lax.GatherDimensionNumbers(
        offset_dims=(), collapsed_slice_dims=(0,), start_index_map=(0,))

    def splat(vec, sp):
      return lax.gather(vec, sp[:, None], dn, (1,),
                        mode=lax.GatherScatterMode.PROMISE_IN_BOUNDS)

    def compute(ch, s, coff, obuf):
      H, W, C, inv = scales[s]
      x, y, _, _, x1f, y1f, x2i, y2i = corner_geom(ch, s)
      wx2 = x - x1f
      wx1 = x2i.astype(jnp.float32) - x
      wy2 = y - y1f
      wy1 = y2i.astype(jnp.float32) - y
      w11 = wx1 * wy1
      w12 = wx1 * wy2
      w21 = wx2 * wy1
      w22 = wx2 * wy2
      gbuf = gbufs[s]

      @plsc.parallel_loop(0, L)
      def row_body(r):
        sp = jnp.full((L,), 0, jnp.int32) + r
        w0 = splat(w11, sp)
        w1 = splat(w12, sp)
        w2 = splat(w21, sp)
        w3 = splat(w22, sp)

        @plsc.parallel_loop(0, C // (2 * L), unroll=4)
        def ch_body(j):
          v0 = gbuf[0 * L + r, pl.ds(j * L, L)]
          v1 = gbuf[1 * L + r, pl.ds(j * L, L)]
          v2 = gbuf[2 * L + r, pl.ds(j * L, L)]
          v3 = gbuf[3 * L + r, pl.ds(j * L, L)]
          lo = w0 * plsc.bitcast(lax.shift_left(v0, 16), jnp.float32)
          lo += w1 * plsc.bitcast(lax.shift_left(v1, 16), jnp.float32)
          lo += w2 * plsc.bitcast(lax.shift_left(v2, 16), jnp.float32)
          lo += w3 * plsc.bitcast(lax.shift_left(v3, 16), jnp.float32)
          hi = w0 * plsc.bitcast(v0 & HI_MASK, jnp.float32)
          hi += w1 * plsc.bitcast(v1 & HI_MASK, jnp.float32)
          hi += w2 * plsc.bitcast(v2 & HI_MASK, jnp.float32)
          hi += w3 * plsc.bitcast(v3 & HI_MASK, jnp.float32)
          obuf[r, pl.ds(coff + 2 * j * L, L)] = lo
          obuf[r, pl.ds(coff + 2 * j * L + L, L)] = hi

    handles = [fire(0, s) for s in range(len(scales))]
    owrites = [None, None]
    for ch in range(n_chunks):
      slot = ch % 2
      if owrites[slot] is not None:
        owrites[slot].wait()
      obuf = obufs[slot]
      coff = 0
      for s in range(len(scales)):
        handles[s].wait()
        compute(ch, s, coff, obuf)
        if ch + 1 < n_chunks:
          handles[s] = fire(ch + 1, s)
        coff += scales[s][2]
      owrites[slot] = pltpu.async_copy(
          obuf, out_hbm.at[pl.ds(base + ch * L, L)], osems[slot])
    for ow in owrites:
      if ow is not None:
        ow.wait()

  return k


def kernel(c, conv_3_3, conv_4_3, conv_5_3):
  B, V, _ = c.shape
  maps = (conv_3_3, conv_4_3, conv_5_3)
  scales = []
  inv = 1.0 / 8.0
  for fm in maps:
    _, C, H, W = fm.shape
    scales.append((H, W, C, inv))
    inv *= 0.5

  cx = c[:, :, 0].reshape(-1)
  cy = c[:, :, 1].reshape(-1)
  tables = []
  for fm in maps:
    C = fm.shape[1]
    t = fm.transpose(0, 2, 3, 1).astype(jnp.bfloat16)
    t = t.reshape(-1, C // 32, 2, 16).swapaxes(2, 3)
    tables.append(
        lax.bitcast_convert_type(t, jnp.int32).reshape(-1, C // 2))

  k = _make_sc_kernel(B, V, tuple(scales))
  out = k(cx, cy, *tables)
  return out.reshape(B, V, out.shape[-1])


# speedup vs baseline: 7.3401x; 4.8969x over previous
import functools

import jax
import jax.numpy as jnp
from jax import lax
from jax.experimental import pallas as pl
from jax.experimental.pallas import tpu as pltpu
from jax.experimental.pallas import tpu_sc as plsc

NC = 2
NS = 16
L = 16
NW = NC * NS


def _make_sc_kernel(B, V, scales):
  ROWS = B * V
  assert ROWS % NW == 0
  rpw = ROWS // NW
  assert rpw % L == 0
  n_chunks = rpw // L
  C_TOT = sum(c for (_, _, c, _) in scales)

  mesh = plsc.VectorSubcoreMesh(
      core_axis_name="c", subcore_axis_name="s",
      num_cores=NC, num_subcores=NS)

  scratch = [
      pltpu.VMEM((rpw,), jnp.float32),
      pltpu.VMEM((rpw,), jnp.float32),
      [pltpu.VMEM((4 * L,), jnp.int32) for _ in scales],
      [pltpu.VMEM((4 * L, c), jnp.float32) for (_, _, c, _) in scales],
      [pltpu.VMEM((L, C_TOT), jnp.float32) for _ in range(2)],
      [pltpu.SemaphoreType.DMA for _ in scales],
      [pltpu.SemaphoreType.DMA for _ in range(2)],
  ]

  @functools.partial(
      pl.kernel,
      mesh=mesh,
      out_type=jax.ShapeDtypeStruct((ROWS, C_TOT), jnp.float32),
      scratch_types=scratch,
  )
  def k(cx_hbm, cy_hbm, t0_hbm, t1_hbm, t2_hbm, out_hbm,
        cx_v, cy_v, idx_vs, gbufs, obufs, gsems, osems):
    tables = (t0_hbm, t1_hbm, t2_hbm)
    wid = lax.axis_index("s") * NC + lax.axis_index("c")
    base = wid * rpw
    batch = base // V

    pltpu.sync_copy(cx_hbm.at[pl.ds(base, rpw)], cx_v)
    pltpu.sync_copy(cy_hbm.at[pl.ds(base, rpw)], cy_v)

    def corner_geom(ch, s):
      H, W, C, inv = scales[s]
      x = cx_v[pl.ds(ch * L, L)] * inv
      y = cy_v[pl.ds(ch * L, L)] * inv
      x1i = x.astype(jnp.int32)
      y1i = y.astype(jnp.int32)
      x1f = x1i.astype(jnp.float32)
      y1f = y1i.astype(jnp.float32)
      one = jnp.full((L,), 1, jnp.int32)
      zero = jnp.full((L,), 0, jnp.int32)
      x2i = x1i + jnp.where(x > x1f, one, zero)
      y2i = y1i + jnp.where(y > y1f, one, zero)
      return x, y, x1i, y1i, x1f, y1f, x2i, y2i

    def fire(ch, s):
      H, W, C, inv = scales[s]
      _, _, x1i, y1i, _, _, x2i, y2i = corner_geom(ch, s)
      idx_v = idx_vs[s]
      r1 = y1i * W + batch * (H * W)
      r2 = y2i * W + batch * (H * W)
      idx_v[pl.ds(0 * L, L)] = r1 + x1i
      idx_v[pl.ds(1 * L, L)] = r2 + x1i
      idx_v[pl.ds(2 * L, L)] = r1 + x2i
      idx_v[pl.ds(3 * L, L)] = r2 + x2i
      return pltpu.async_copy(tables[s].at[idx_v], gbufs[s], gsems[s])

    dn = lax.GatherDimensionNumbers(
        offset_dims=(), collapsed_slice_dims=(0,), start_index_map=(0,))

    def splat(vec, sp):
      return lax.gather(vec, sp[:, None], dn, (1,),
                        mode=lax.GatherScatterMode.PROMISE_IN_BOUNDS)

    def compute(ch, s, coff, obuf):
      H, W, C, inv = scales[s]
      x, y, _, _, x1f, y1f, x2i, y2i = corner_geom(ch, s)
      wx2 = x - x1f
      wx1 = x2i.astype(jnp.float32) - x
      wy2 = y - y1f
      wy1 = y2i.astype(jnp.float32) - y
      w11 = wx1 * wy1
      w12 = wx1 * wy2
      w21 = wx2 * wy1
      w22 = wx2 * wy2
      gbuf = gbufs[s]

      @plsc.parallel_loop(0, L)
      def row_body(r):
        sp = jnp.full((L,), 0, jnp.int32) + r
        w0 = splat(w11, sp)
        w1 = splat(w12, sp)
        w2 = splat(w21, sp)
        w3 = splat(w22, sp)

        @plsc.parallel_loop(0, C // L, unroll=4)
        def ch_body(j):
          acc = w0 * gbuf[0 * L + r, pl.ds(j * L, L)]
          acc += w1 * gbuf[1 * L + r, pl.ds(j * L, L)]
          acc += w2 * gbuf[2 * L + r, pl.ds(j * L, L)]
          acc += w3 * gbuf[3 * L + r, pl.ds(j * L, L)]
          obuf[r, pl.ds(coff + j * L, L)] = acc

    handles = [fire(0, s) for s in range(len(scales))]
    owrites = [None, None]
    for ch in range(n_chunks):
      slot = ch % 2
      if owrites[slot] is not None:
        owrites[slot].wait()
      obuf = obufs[slot]
      coff = 0
      for s in range(len(scales)):
        handles[s].wait()
        compute(ch, s, coff, obuf)
        if ch + 1 < n_chunks:
          handles[s] = fire(ch + 1, s)
        coff += scales[s][2]
      owrites[slot] = pltpu.async_copy(
          obuf, out_hbm.at[pl.ds(base + ch * L, L)], osems[slot])
    for ow in owrites:
      if ow is not None:
        ow.wait()

  return k


def kernel(c, conv_3_3, conv_4_3, conv_5_3):
  B, V, _ = c.shape
  maps = (conv_3_3, conv_4_3, conv_5_3)
  scales = []
  inv = 1.0 / 8.0
  for fm in maps:
    _, C, H, W = fm.shape
    scales.append((H, W, C, inv))
    inv *= 0.5

  cx = c[:, :, 0].reshape(-1)
  cy = c[:, :, 1].reshape(-1)
  tables = [fm.transpose(0, 2, 3, 1).reshape(-1, fm.shape[1]) for fm in maps]

  k = _make_sc_kernel(B, V, tuple(scales))
  out = k(cx, cy, *tables)
  return out.reshape(B, V, out.shape[-1])


# speedup vs baseline: 7.3883x; 1.0066x over previous
import functools

import jax
import jax.numpy as jnp
from jax import lax
from jax.experimental import pallas as pl
from jax.experimental.pallas import tpu as pltpu
from jax.experimental.pallas import tpu_sc as plsc

NC = 2
NS = 16
L = 16
NW = NC * NS


def _make_sc_kernel(B, V, scales):
  ROWS = B * V
  assert ROWS % NW == 0
  rpw = ROWS // NW
  assert rpw % L == 0
  n_chunks = rpw // L
  C_TOT = sum(c for (_, _, c, _) in scales)

  mesh = plsc.VectorSubcoreMesh(
      core_axis_name="c", subcore_axis_name="s",
      num_cores=NC, num_subcores=NS)

  scratch = [
      pltpu.VMEM((2 * rpw,), jnp.float32),
      [pltpu.VMEM((4 * L,), jnp.int32) for _ in scales],
      [pltpu.VMEM((4 * L, c), jnp.float32) for (_, _, c, _) in scales],
      [pltpu.VMEM((L, C_TOT), jnp.float32) for _ in range(2)],
      [pltpu.SemaphoreType.DMA for _ in scales],
      [pltpu.SemaphoreType.DMA for _ in range(2)],
  ]

  @functools.partial(
      pl.kernel,
      mesh=mesh,
      out_type=jax.ShapeDtypeStruct((ROWS, C_TOT), jnp.float32),
      scratch_types=scratch,
  )
  def k(cxy_hbm, t0_hbm, t1_hbm, t2_hbm, out_hbm,
        cxy_v, idx_vs, gbufs, obufs, gsems, osems):
    tables = (t0_hbm, t1_hbm, t2_hbm)
    wid = lax.axis_index("s") * NC + lax.axis_index("c")
    base = wid * rpw
    batch = base // V

    pltpu.sync_copy(cxy_hbm.at[pl.ds(wid * 2 * rpw, 2 * rpw)], cxy_v)

    def corner_geom(ch, s):
      H, W, C, inv = scales[s]
      x = cxy_v[pl.ds(ch * L, L)] * inv
      y = cxy_v[pl.ds(rpw + ch * L, L)] * inv
      x1i = x.astype(jnp.int32)
      y1i = y.astype(jnp.int32)
      x1f = x1i.astype(jnp.float32)
      y1f = y1i.astype(jnp.float32)
      one = jnp.full((L,), 1, jnp.int32)
      zero = jnp.full((L,), 0, jnp.int32)
      x2i = x1i + jnp.where(x > x1f, one, zero)
      y2i = y1i + jnp.where(y > y1f, one, zero)
      return x, y, x1i, y1i, x1f, y1f, x2i, y2i

    def fire(ch, s):
      H, W, C, inv = scales[s]
      _, _, x1i, y1i, _, _, x2i, y2i = corner_geom(ch, s)
      idx_v = idx_vs[s]
      r1 = y1i * W + batch * (H * W)
      r2 = y2i * W + batch * (H * W)
      idx_v[pl.ds(0 * L, L)] = r1 + x1i
      idx_v[pl.ds(1 * L, L)] = r2 + x1i
      idx_v[pl.ds(2 * L, L)] = r1 + x2i
      idx_v[pl.ds(3 * L, L)] = r2 + x2i
      return pltpu.async_copy(tables[s].at[idx_v], gbufs[s], gsems[s])

    dn = lax.GatherDimensionNumbers(
        offset_dims=(), collapsed_slice_dims=(0,), start_index_map=(0,))

    def splat(vec, sp):
      return lax.gather(vec, sp[:, None], dn, (1,),
                        mode=lax.GatherScatterMode.PROMISE_IN_BOUNDS)

    def compute(ch, s, coff, obuf):
      H, W, C, inv = scales[s]
      x, y, _, _, x1f, y1f, x2i, y2i = corner_geom(ch, s)
      wx2 = x - x1f
      wx1 = x2i.astype(jnp.float32) - x
      wy2 = y - y1f
      wy1 = y2i.astype(jnp.float32) - y
      w11 = wx1 * wy1
      w12 = wx1 * wy2
      w21 = wx2 * wy1
      w22 = wx2 * wy2
      gbuf = gbufs[s]

      @plsc.parallel_loop(0, L)
      def row_body(r):
        sp = jnp.full((L,), 0, jnp.int32) + r
        w0 = splat(w11, sp)
        w1 = splat(w12, sp)
        w2 = splat(w21, sp)
        w3 = splat(w22, sp)

        @plsc.parallel_loop(0, C // L, unroll=8)
        def ch_body(j):
          acc = w0 * gbuf[0 * L + r, pl.ds(j * L, L)]
          acc += w1 * gbuf[1 * L + r, pl.ds(j * L, L)]
          acc += w2 * gbuf[2 * L + r, pl.ds(j * L, L)]
          acc += w3 * gbuf[3 * L + r, pl.ds(j * L, L)]
          obuf[r, pl.ds(coff + j * L, L)] = acc

    handles = [fire(0, s) for s in range(len(scales))]
    owrites = [None, None]
    for ch in range(n_chunks):
      slot = ch % 2
      if owrites[slot] is not None:
        owrites[slot].wait()
      obuf = obufs[slot]
      coff = 0
      for s in range(len(scales)):
        handles[s].wait()
        compute(ch, s, coff, obuf)
        if ch + 1 < n_chunks:
          handles[s] = fire(ch + 1, s)
        coff += scales[s][2]
      owrites[slot] = pltpu.async_copy(
          obuf, out_hbm.at[pl.ds(base + ch * L, L)], osems[slot])
    for ow in owrites:
      if ow is not None:
        ow.wait()

  return k


def kernel(c, conv_3_3, conv_4_3, conv_5_3):
  B, V, _ = c.shape
  maps = (conv_3_3, conv_4_3, conv_5_3)
  scales = []
  inv = 1.0 / 8.0
  for fm in maps:
    _, C, H, W = fm.shape
    scales.append((H, W, C, inv))
    inv *= 0.5

  rpw = (B * V) // NW
  cxy = c.reshape(NW, rpw, 2).swapaxes(1, 2).reshape(-1)
  tables = [fm.transpose(0, 2, 3, 1).reshape(-1, fm.shape[1]) for fm in maps]

  k = _make_sc_kernel(B, V, tuple(scales))
  out = k(cxy, *tables)
  return out.reshape(B, V, out.shape[-1])
